# baseline (speedup 1.0000x reference)
import functools
import math

import jax
import jax.numpy as jnp
from jax import lax
from jax.experimental import pallas as pl
from jax.experimental.pallas import tpu as pltpu
from jax.experimental.pallas import tpu_sc as plsc

_SIZE = 32000
_N_TOK = 2048
_SMOOTHING = 0.1
_CONF = 1.0 - _SMOOTHING
_S = _SMOOTHING / (_SIZE - 2)
_C1 = _CONF * math.log(_CONF) + (_SIZE - 2) * _S * math.log(_S)

_CSC = 4864
_C0 = _SIZE - _CSC

_R = 256
_W = 6784

_LANES = 16
_NC = 2
_NS = 16
_NW = _NC * _NS
_BPW = _N_TOK // _NW


def _tc_body(tcol_ref, trow_ref, x_ref, out_ref):
    i = pl.program_id(0)
    j = pl.program_id(1)

    x = x_ref[...]
    tcol = tcol_ref[...]
    trow = trow_ref[...]

    has_nz = jnp.any(trow != 0)
    has_z = jnp.any(trow == 0)

    rid = lax.broadcasted_iota(jnp.int32, (_R, 1), 0) + i * _R
    surv = (tcol != 0) \
        & jnp.logical_not((rid == 0) & has_nz) \
        & jnp.logical_not((rid == 1) & has_z)
    survf = surv.astype(jnp.float32)

    rs = jnp.sum(x, axis=1, keepdims=True)
    partial = (-_S) * jnp.sum(survf * rs)
    partial = partial + jnp.where(
        j == 0, _S * jnp.sum(survf * x[:, 0:1]), 0.0)

    first = (i == 0) & (j == 0)

    @pl.when(first)
    def _():
        out_ref[...] = jnp.reshape(partial, (1, 1))

    @pl.when(jnp.logical_not(first))
    def _():
        out_ref[...] = out_ref[...] + partial


def _tc_call(x, tcol, trow):
    return pl.pallas_call(
        _tc_body,
        grid=(_N_TOK // _R, _C0 // _W),
        in_specs=[
            pl.BlockSpec((_R, 1), lambda i, j: (i, 0)),
            pl.BlockSpec((1, _N_TOK), lambda i, j: (0, 0)),
            pl.BlockSpec((_R, _W), lambda i, j: (i, j)),
        ],
        out_specs=pl.BlockSpec((1, 1), lambda i, j: (0, 0)),
        out_shape=jax.ShapeDtypeStruct((1, 1), jnp.float32),
    )(tcol, trow, x)


_sc_mesh = plsc.VectorSubcoreMesh(core_axis_name="c", subcore_axis_name="s")


@functools.partial(
    pl.kernel,
    mesh=_sc_mesh,
    compiler_params=pltpu.CompilerParams(needs_layout_passes=False),
    out_type=jax.ShapeDtypeStruct((_NW, _LANES), jnp.float32),
    scratch_types=[
        pltpu.VMEM((_BPW,), jnp.int32),
        pltpu.VMEM((_BPW, 8, 128), jnp.float32),
        pltpu.VMEM((_N_TOK,), jnp.int32),
        pltpu.VMEM((_LANES,), jnp.float32),
        pltpu.VMEM((_LANES,), jnp.int32),
        pltpu.VMEM((_BPW,), jnp.float32),
        pltpu.VMEM((8, _CSC), jnp.float32),
        pltpu.SemaphoreType.DMA,
    ],
)
def _sc_part(x_hbm, tgt_hbm, out_hbm,
             tgt_v, rows_v, tfull_v, part_v, zbuf_v, sw_v, sbuf_v, sem):
    wid = lax.axis_index("s") * _NC + lax.axis_index("c")
    base = wid * _BPW

    pltpu.sync_copy(tgt_hbm.at[pl.ds(base, _BPW)], tgt_v)
    pltpu.sync_copy(tgt_hbm, tfull_v)

    copies = []
    for k in range(_BPW // _LANES):
        t16 = tgt_v[pl.ds(k * _LANES, _LANES)]
        for j in range(_LANES):
            r = k * _LANES + j
            cstart = pl.multiple_of(
                jnp.bitwise_and(t16[j], jnp.int32(~127)), 128)
            copies.append(pltpu.async_copy(
                x_hbm.at[pl.ds(base + (r // 8) * 8, 8), pl.ds(cstart, 128)],
                rows_v.at[r], sem))
    for c in copies:
        c.wait()

    lane = lax.broadcasted_iota(jnp.int32, (_LANES,), 0)

    def _scan(k, z):
        t16 = tfull_v[pl.ds(k * _LANES, _LANES)]
        return z + jnp.where(t16 == 0, 1, 0).astype(jnp.int32)
    zcnt = lax.fori_loop(0, _N_TOK // _LANES, _scan,
                         jnp.zeros((_LANES,), jnp.int32))
    for step in (1, 2, 4, 8):
        zbuf_v[...] = zcnt
        zcnt = zcnt + plsc.load_gather(zbuf_v, [jnp.bitwise_xor(lane, step)])
    has_z = zcnt > 0

    t_first = tgt_v[pl.ds(0, _LANES)]
    cm = ((lane == 0) & (t_first != 0)) \
        | ((lane == 1) & (t_first != 0) & has_z)
    cm = cm & (wid == 0)

    zero16 = jnp.zeros((_LANES,), jnp.float32)
    acc = zero16
    cnt = zero16
    sub16 = jnp.bitwise_and(lane, 7)
    for k in range(_BPW // _LANES):
        t16 = tgt_v[pl.ds(k * _LANES, _LANES)]
        col16 = jnp.bitwise_and(t16, 127)
        row16 = k * _LANES + lane
        g16 = plsc.load_gather(rows_v, [row16, sub16, col16])
        sw16 = jnp.where(t16 != 0, 1.0, 0.0)
        if k == 0:
            sw16 = sw16 - jnp.where(cm, 1.0, 0.0)
        sw_v[pl.ds(k * _LANES, _LANES)] = sw16
        acc = acc + g16 * sw16
        cnt = cnt + sw16

    csum = zero16
    for g in range(_BPW // 8):
        pltpu.sync_copy(
            x_hbm.at[pl.ds(base + g * 8, 8), pl.ds(_C0, _CSC)], sbuf_v)
        for j in range(8):
            r = g * 8 + j

            def _rbody(i, a, j=j):
                o = i * (16 * _LANES)
                for u in range(16):
                    a = a + sbuf_v[j, pl.ds(o + u * _LANES, _LANES)]
                return a

            rsum = lax.fori_loop(0, _CSC // (16 * _LANES), _rbody, zero16)
            wv = plsc.load_gather(sw_v, [jnp.zeros_like(lane) + r])
            csum = csum + wv * rsum

    part_v[...] = _C1 * cnt - (_CONF - _S) * acc - _S * csum
    pltpu.sync_copy(part_v, out_hbm.at[wid])


def kernel(x, target):
    tcol = target.reshape(_N_TOK, 1)
    trow = target.reshape(1, _N_TOK)
    part = _sc_part(x, target)
    a = _tc_call(x, tcol, trow)
    return a[0, 0] + jnp.sum(part)


# speedup vs baseline: 1.0165x; 1.0165x over previous
import functools
import math

import jax
import jax.numpy as jnp
from jax import lax
from jax.experimental import pallas as pl
from jax.experimental.pallas import tpu as pltpu
from jax.experimental.pallas import tpu_sc as plsc

_SIZE = 32000
_N_TOK = 2048
_SMOOTHING = 0.1
_CONF = 1.0 - _SMOOTHING
_S = _SMOOTHING / (_SIZE - 2)
_C1 = _CONF * math.log(_CONF) + (_SIZE - 2) * _S * math.log(_S)

_R = 128
_W = 16000

_LANES = 16
_NC = 2
_NS = 16
_NW = _NC * _NS
_BPW = _N_TOK // _NW


def _tc_body(tcol_ref, trow_ref, x_ref, out_ref):
    i = pl.program_id(0)
    j = pl.program_id(1)

    x = x_ref[...]
    tcol = tcol_ref[...]
    trow = trow_ref[...]

    has_nz = jnp.any(trow != 0)
    has_z = jnp.any(trow == 0)

    rid = lax.broadcasted_iota(jnp.int32, (_R, 1), 0) + i * _R
    surv = (tcol != 0) \
        & jnp.logical_not((rid == 0) & has_nz) \
        & jnp.logical_not((rid == 1) & has_z)
    survf = surv.astype(jnp.float32)

    rs = jnp.sum(x, axis=1, keepdims=True)
    partial = (-_S) * jnp.sum(survf * rs)
    partial = partial + jnp.where(
        j == 0, _S * jnp.sum(survf * x[:, 0:1]), 0.0)

    first = (i == 0) & (j == 0)

    @pl.when(first)
    def _():
        out_ref[...] = jnp.reshape(partial, (1, 1))

    @pl.when(jnp.logical_not(first))
    def _():
        out_ref[...] = out_ref[...] + partial


def _tc_call(x, tcol, trow):
    return pl.pallas_call(
        _tc_body,
        grid=(_N_TOK // _R, _SIZE // _W),
        in_specs=[
            pl.BlockSpec((_R, 1), lambda i, j: (i, 0)),
            pl.BlockSpec((1, _N_TOK), lambda i, j: (0, 0)),
            pl.BlockSpec((_R, _W), lambda i, j: (i, j)),
        ],
        out_specs=pl.BlockSpec((1, 1), lambda i, j: (0, 0)),
        out_shape=jax.ShapeDtypeStruct((1, 1), jnp.float32),
    )(tcol, trow, x)


_sc_mesh = plsc.VectorSubcoreMesh(core_axis_name="c", subcore_axis_name="s")


@functools.partial(
    pl.kernel,
    mesh=_sc_mesh,
    compiler_params=pltpu.CompilerParams(needs_layout_passes=False),
    out_type=jax.ShapeDtypeStruct((_NW, _LANES), jnp.float32),
    scratch_types=[
        pltpu.VMEM((_BPW,), jnp.int32),
        pltpu.VMEM((_BPW, 8, 128), jnp.float32),
        pltpu.VMEM((_N_TOK,), jnp.int32),
        pltpu.VMEM((_LANES,), jnp.float32),
        pltpu.VMEM((_LANES,), jnp.int32),
        pltpu.SemaphoreType.DMA,
    ],
)
def _sc_part(x_hbm, tgt_hbm, out_hbm,
             tgt_v, rows_v, tfull_v, part_v, zbuf_v, sem):
    wid = lax.axis_index("s") * _NC + lax.axis_index("c")
    base = wid * _BPW

    pltpu.sync_copy(tgt_hbm.at[pl.ds(base, _BPW)], tgt_v)
    pltpu.sync_copy(tgt_hbm, tfull_v)

    copies = []
    for k in range(_BPW // _LANES):
        t16 = tgt_v[pl.ds(k * _LANES, _LANES)]
        for j in range(_LANES):
            r = k * _LANES + j
            cstart = pl.multiple_of(
                jnp.bitwise_and(t16[j], jnp.int32(~127)), 128)
            copies.append(pltpu.async_copy(
                x_hbm.at[pl.ds(base + (r // 8) * 8, 8), pl.ds(cstart, 128)],
                rows_v.at[r], sem))
    for c in copies:
        c.wait()

    lane = lax.broadcasted_iota(jnp.int32, (_LANES,), 0)

    def _scan(k, z):
        t16 = tfull_v[pl.ds(k * _LANES, _LANES)]
        return z + jnp.where(t16 == 0, 1, 0).astype(jnp.int32)
    zcnt = lax.fori_loop(0, _N_TOK // _LANES, _scan,
                         jnp.zeros((_LANES,), jnp.int32))
    for step in (1, 2, 4, 8):
        zbuf_v[...] = zcnt
        zcnt = zcnt + plsc.load_gather(zbuf_v, [jnp.bitwise_xor(lane, step)])
    has_z = zcnt > 0

    zero16 = jnp.zeros((_LANES,), jnp.float32)
    acc = zero16
    cnt = zero16
    g_first = zero16
    t_first = jnp.zeros((_LANES,), jnp.int32)
    sub16 = jnp.bitwise_and(lane, 7)
    for k in range(_BPW // _LANES):
        t16 = tgt_v[pl.ds(k * _LANES, _LANES)]
        col16 = jnp.bitwise_and(t16, 127)
        row16 = k * _LANES + lane
        g16 = plsc.load_gather(rows_v, [row16, sub16, col16])
        m = t16 != 0
        acc = acc + jnp.where(m, g16, 0.0)
        cnt = cnt + jnp.where(m, 1.0, 0.0)
        if k == 0:
            g_first = g16
            t_first = t16

    cm = ((lane == 0) & (t_first != 0)) \
        | ((lane == 1) & (t_first != 0) & has_z)
    cm = cm & (wid == 0)
    acc = acc - jnp.where(cm, g_first, 0.0)
    cnt = cnt - jnp.where(cm, 1.0, 0.0)

    part_v[...] = _C1 * cnt - (_CONF - _S) * acc
    pltpu.sync_copy(part_v, out_hbm.at[wid])


def kernel(x, target):
    tcol = target.reshape(_N_TOK, 1)
    trow = target.reshape(1, _N_TOK)
    part = _sc_part(x, target)
    a = _tc_call(x, tcol, trow)
    return a[0, 0] + jnp.sum(part)
